# Initial kernel scaffold; baseline (speedup 1.0000x reference)
import jax
import jax.numpy as jnp
from jax.experimental import pallas as pl


def kernel(input, emb, W1, b1, W2, b2):
    raise NotImplementedError("write your pallas kernel here")


# speedup vs baseline: 2.0567x; 2.0567x over previous
import functools

import jax
import jax.numpy as jnp
from jax import lax
from jax.experimental import pallas as pl
from jax.experimental.pallas import tpu as pltpu
from jax.experimental.pallas import tpu_sc as plsc

V, D, H = 1_000_000, 64, 100
L, B = 200, 16384

NC, NS = 2, 16
NW = NC * NS
BPW = B // NW
GCH = 128
NG = BPW // GCH
CL = 40
NCH = L // CL

_mesh = plsc.VectorSubcoreMesh(core_axis_name="c", subcore_axis_name="s")


@functools.partial(
    pl.kernel,
    out_type=jax.ShapeDtypeStruct((B, D), jnp.float32),
    mesh=_mesh,
    scratch_types=[
        pltpu.VMEM((CL, BPW), jnp.int32),
        pltpu.VMEM((BPW, D), jnp.float32),
        pltpu.SemaphoreType.DMA,
    ],
    compiler_params=pltpu.CompilerParams(use_tc_tiling_on_sc=False),
)
def _sc_pool(inp_hbm, emb_hbm, out_hbm, idx_v, acc_v, sem):
    wid = lax.axis_index("s") * NC + lax.axis_index("c")
    base = wid * BPW

    zeros16 = jnp.zeros((16,), jnp.float32)

    def zero_body(i, carry):
        for c in range(D // 16):
            acc_v[i, pl.ds(c * 16, 16)] = zeros16
        return carry

    lax.fori_loop(0, BPW, zero_body, 0)

    for ci in range(NCH):
        pltpu.sync_copy(
            inp_hbm.at[pl.ds(ci * CL, CL), pl.ds(base, BPW)], idx_v
        )

        def l_body(l, carry):
            cps = []
            for g in range(NG):
                cps.append(
                    pltpu.async_copy(
                        emb_hbm.at[idx_v.at[l, pl.ds(g * GCH, GCH)]],
                        acc_v.at[pl.ds(g * GCH, GCH), :],
                        sem,
                        add=True,
                    )
                )
            for cp in cps:
                cp.wait()
            return carry

        lax.fori_loop(0, CL, l_body, 0)

    pltpu.sync_copy(acc_v, out_hbm.at[pl.ds(base, BPW)])


def _mlp_body(x_ref, w1_ref, b1_ref, w2_ref, b2_ref, o_ref):
    x = x_ref[...] * (1.0 / L)
    h = jnp.maximum(
        jnp.dot(x, w1_ref[...], preferred_element_type=jnp.float32)
        + b1_ref[...],
        0.0,
    )
    o_ref[...] = (
        jnp.dot(h, w2_ref[...], preferred_element_type=jnp.float32)
        + b2_ref[...]
    )


_BM = 2048


def _mlp(pooled_sum, W1, b1, W2, b2):
    return pl.pallas_call(
        _mlp_body,
        grid=(B // _BM,),
        in_specs=[
            pl.BlockSpec((_BM, D), lambda i: (i, 0)),
            pl.BlockSpec((D, H), lambda i: (0, 0)),
            pl.BlockSpec((1, H), lambda i: (0, 0)),
            pl.BlockSpec((H, 1), lambda i: (0, 0)),
            pl.BlockSpec((1, 1), lambda i: (0, 0)),
        ],
        out_specs=pl.BlockSpec((_BM, 1), lambda i: (i, 0)),
        out_shape=jax.ShapeDtypeStruct((B, 1), jnp.float32),
    )(pooled_sum, W1, b1.reshape(1, H), W2, b2.reshape(1, 1))


def kernel(input, emb, W1, b1, W2, b2):
    pooled_sum = _sc_pool(input, emb)
    out = _mlp(pooled_sum, W1, b1, W2, b2)
    return out[:, :, None]


# speedup vs baseline: 2.3200x; 1.1280x over previous
import functools

import jax
import jax.numpy as jnp
from jax import lax
from jax.experimental import pallas as pl
from jax.experimental.pallas import tpu as pltpu
from jax.experimental.pallas import tpu_sc as plsc

V, D, H = 1_000_000, 64, 100
L, B = 200, 16384

NC, NS = 2, 16
NW = NC * NS
BPW = B // NW
GCH = 128
NG = BPW // GCH
CL = 40
NCH = L // CL

_mesh = plsc.VectorSubcoreMesh(core_axis_name="c", subcore_axis_name="s")


@functools.partial(
    pl.kernel,
    out_type=jax.ShapeDtypeStruct((B, D), jnp.float32),
    mesh=_mesh,
    scratch_types=[
        pltpu.VMEM((CL, BPW), jnp.int32),
        pltpu.VMEM((BPW, D), jnp.float32),
        pltpu.SemaphoreType.DMA,
    ],
    compiler_params=pltpu.CompilerParams(use_tc_tiling_on_sc=False),
)
def _sc_pool(inp_hbm, emb_hbm, out_hbm, idx_v, acc_v, sem):
    wid = lax.axis_index("s") * NC + lax.axis_index("c")
    base = wid * BPW

    zeros16 = jnp.zeros((16,), jnp.float32)

    def zero_body(i, carry):
        for c in range(D // 16):
            acc_v[i, pl.ds(c * 16, 16)] = zeros16
        return carry

    lax.fori_loop(0, BPW, zero_body, 0)

    DEPTH = 4

    def fire(l):
        for g in range(NG):
            pltpu.async_copy(
                emb_hbm.at[idx_v.at[l, pl.ds(g * GCH, GCH)]],
                acc_v.at[pl.ds(g * GCH, GCH), :],
                sem,
                add=True,
            )

    def drain_one(l):
        for g in range(NG):
            pltpu.make_async_copy(
                emb_hbm.at[idx_v.at[l, pl.ds(g * GCH, GCH)]],
                acc_v.at[pl.ds(g * GCH, GCH), :],
                sem,
            ).wait()

    for ci in range(NCH):
        pltpu.sync_copy(
            inp_hbm.at[pl.ds(ci * CL, CL), pl.ds(base, BPW)], idx_v
        )

        for l in range(DEPTH):
            fire(l)

        def l_body(l, carry):
            fire(l)
            drain_one(l - DEPTH)
            return carry

        lax.fori_loop(DEPTH, CL, l_body, 0)

        for l in range(CL - DEPTH, CL):
            drain_one(l)

    pltpu.sync_copy(acc_v, out_hbm.at[pl.ds(base, BPW)])


def _mlp_body(x_ref, w1_ref, b1_ref, w2_ref, b2_ref, o_ref):
    x = x_ref[...] * (1.0 / L)
    h = jnp.maximum(
        jnp.dot(x, w1_ref[...], preferred_element_type=jnp.float32)
        + b1_ref[...],
        0.0,
    )
    o_ref[...] = (
        jnp.dot(h, w2_ref[...], preferred_element_type=jnp.float32)
        + b2_ref[...]
    )


_BM = 2048


def _mlp(pooled_sum, W1, b1, W2, b2):
    return pl.pallas_call(
        _mlp_body,
        grid=(B // _BM,),
        in_specs=[
            pl.BlockSpec((_BM, D), lambda i: (i, 0)),
            pl.BlockSpec((D, H), lambda i: (0, 0)),
            pl.BlockSpec((1, H), lambda i: (0, 0)),
            pl.BlockSpec((H, 1), lambda i: (0, 0)),
            pl.BlockSpec((1, 1), lambda i: (0, 0)),
        ],
        out_specs=pl.BlockSpec((_BM, 1), lambda i: (i, 0)),
        out_shape=jax.ShapeDtypeStruct((B, 1), jnp.float32),
    )(pooled_sum, W1, b1.reshape(1, H), W2, b2.reshape(1, 1))


def kernel(input, emb, W1, b1, W2, b2):
    pooled_sum = _sc_pool(input, emb)
    out = _mlp(pooled_sum, W1, b1, W2, b2)
    return out[:, :, None]


# speedup vs baseline: 2.3227x; 1.0012x over previous
import functools

import jax
import jax.numpy as jnp
from jax import lax
from jax.experimental import pallas as pl
from jax.experimental.pallas import tpu as pltpu
from jax.experimental.pallas import tpu_sc as plsc

V, D, H = 1_000_000, 64, 100
L, B = 200, 16384

NC, NS = 2, 16
NW = NC * NS
BPW = B // NW
GCH = 512
NG = BPW // GCH
CL = 40
NCH = L // CL

_mesh = plsc.VectorSubcoreMesh(core_axis_name="c", subcore_axis_name="s")


@functools.partial(
    pl.kernel,
    out_type=jax.ShapeDtypeStruct((B, D), jnp.float32),
    mesh=_mesh,
    scratch_types=[
        pltpu.VMEM((CL, BPW), jnp.int32),
        pltpu.VMEM((BPW, D), jnp.float32),
        pltpu.SemaphoreType.DMA,
    ],
    compiler_params=pltpu.CompilerParams(use_tc_tiling_on_sc=False),
)
def _sc_pool(inp_hbm, emb_hbm, out_hbm, idx_v, acc_v, sem):
    wid = lax.axis_index("s") * NC + lax.axis_index("c")
    base = wid * BPW

    zeros16 = jnp.zeros((16,), jnp.float32)

    def zero_body(i, carry):
        for c in range(D // 16):
            acc_v[i, pl.ds(c * 16, 16)] = zeros16
        return carry

    lax.fori_loop(0, BPW, zero_body, 0)

    DEPTH = 4

    def fire(l):
        for g in range(NG):
            pltpu.async_copy(
                emb_hbm.at[idx_v.at[l, pl.ds(g * GCH, GCH)]],
                acc_v.at[pl.ds(g * GCH, GCH), :],
                sem,
                add=True,
            )

    def drain_one(l):
        for g in range(NG):
            pltpu.make_async_copy(
                emb_hbm.at[idx_v.at[l, pl.ds(g * GCH, GCH)]],
                acc_v.at[pl.ds(g * GCH, GCH), :],
                sem,
            ).wait()

    for ci in range(NCH):
        pltpu.sync_copy(
            inp_hbm.at[pl.ds(ci * CL, CL), pl.ds(base, BPW)], idx_v
        )

        for l in range(DEPTH):
            fire(l)

        def l_body(l, carry):
            fire(l)
            drain_one(l - DEPTH)
            return carry

        lax.fori_loop(DEPTH, CL, l_body, 0)

        for l in range(CL - DEPTH, CL):
            drain_one(l)

    pltpu.sync_copy(acc_v, out_hbm.at[pl.ds(base, BPW)])


def _mlp_body(x_ref, w1_ref, b1_ref, w2_ref, b2_ref, o_ref):
    x = x_ref[...] * (1.0 / L)
    h = jnp.maximum(
        jnp.dot(x, w1_ref[...], preferred_element_type=jnp.float32)
        + b1_ref[...],
        0.0,
    )
    o_ref[...] = (
        jnp.dot(h, w2_ref[...], preferred_element_type=jnp.float32)
        + b2_ref[...]
    )


_BM = 2048


def _mlp(pooled_sum, W1, b1, W2, b2):
    return pl.pallas_call(
        _mlp_body,
        grid=(B // _BM,),
        in_specs=[
            pl.BlockSpec((_BM, D), lambda i: (i, 0)),
            pl.BlockSpec((D, H), lambda i: (0, 0)),
            pl.BlockSpec((1, H), lambda i: (0, 0)),
            pl.BlockSpec((H, 1), lambda i: (0, 0)),
            pl.BlockSpec((1, 1), lambda i: (0, 0)),
        ],
        out_specs=pl.BlockSpec((_BM, 1), lambda i: (i, 0)),
        out_shape=jax.ShapeDtypeStruct((B, 1), jnp.float32),
    )(pooled_sum, W1, b1.reshape(1, H), W2, b2.reshape(1, 1))


def kernel(input, emb, W1, b1, W2, b2):
    pooled_sum = _sc_pool(input, emb)
    out = _mlp(pooled_sum, W1, b1, W2, b2)
    return out[:, :, None]
